# baseline (speedup 1.0000x reference)
import jax
import jax.numpy as jnp
from jax.experimental import pallas as pl
from jax.experimental.pallas import tpu as pltpu
from jax.experimental.pallas import tpu_sc as plsc

T = 512
D = 2048
F = 7168
E = 8
EPAD = 128
BLK_F = 256
NF = F // BLK_F
BLK_M = 128
NB_MAX = 16
R_PAD = NB_MAX * BLK_M


def _gate_body(logits_ref, ew_ref):
    lane = jax.lax.broadcasted_iota(jnp.int32, (T, EPAD), 1)
    valid = lane < E
    neg_inf = jnp.float32(-jnp.inf)
    logits = jnp.where(valid, logits_ref[...], neg_inf)
    lmax = jnp.max(logits, axis=1, keepdims=True)
    unnorm = jnp.exp(logits - lmax)
    p = unnorm / jnp.sum(unnorm, axis=1, keepdims=True)
    p = jnp.where(valid, p, neg_inf)
    m1 = jnp.max(p, axis=1, keepdims=True)
    i1 = jnp.min(jnp.where(p == m1, lane, EPAD), axis=1, keepdims=True)
    oh1 = lane == i1
    p2 = jnp.where(oh1, neg_inf, p)
    m2 = jnp.max(p2, axis=1, keepdims=True)
    i2 = jnp.min(jnp.where(p2 == m2, lane, EPAD), axis=1, keepdims=True)
    oh2 = lane == i2
    denom = m1 + m2
    ew_ref[...] = (jnp.where(oh1, m1, 0.0) + jnp.where(oh2, m2, 0.0)) / denom


def _gate(logits_pad):
    return pl.pallas_call(
        _gate_body,
        out_shape=jax.ShapeDtypeStruct((T, EPAD), jnp.float32),
    )(logits_pad)


def _routing_metadata(ew_t):
    i32 = jnp.int32
    rw, sel = jax.lax.top_k(ew_t[:, :E], 2)
    flat_sel = sel.reshape(-1)
    order = jnp.argsort(flat_sel, stable=True)
    sorted_e = flat_sel[order]
    counts = jnp.sum(flat_sel[None, :] == jnp.arange(E)[:, None], axis=1)
    nblk = (counts + BLK_M - 1) // BLK_M
    blk_off = jnp.concatenate([jnp.zeros(1, i32),
                               jnp.cumsum(nblk)[:-1].astype(i32)])
    off = jnp.concatenate([jnp.zeros(1, i32),
                           jnp.cumsum(counts)[:-1].astype(i32)])
    j = jnp.arange(2 * T, dtype=i32)
    rank = j - off[sorted_e]
    padded_pos = blk_off[sorted_e] * BLK_M + rank
    row_src = jnp.zeros(R_PAD, i32).at[padded_pos].set((order // 2).astype(i32))
    row_w = jnp.zeros(R_PAD, jnp.float32).at[padded_pos].set(rw.reshape(-1)[order])
    pos_flat = jnp.zeros(2 * T, i32).at[order].set(padded_pos.astype(i32))
    pos0, pos1 = pos_flat[0::2], pos_flat[1::2]
    nb_actual = jnp.sum(nblk).astype(i32)
    c = jnp.arange(NB_MAX, dtype=i32)
    raw_bexp = jnp.searchsorted(jnp.cumsum(nblk), c, side="right").astype(i32)
    last_e = sorted_e[-1].astype(i32)
    bexp = jnp.where(c < nb_actual, jnp.minimum(raw_bexp, E - 1), last_e)
    bact = (c < nb_actual).astype(i32)
    return row_src, row_w[:, None], pos0, pos1, bexp, bact


def _ffn_body(bexp_ref, bact_ref, xs_ref, w1_ref, w3_ref, w2_ref, roww_ref,
              o_ref, acc_ref, w1c_ref, w3c_ref, w2c_ref):
    f = pl.program_id(0)
    b = pl.program_id(1)

    @pl.when(bact_ref[b] == 1)
    def _():
        prev = bexp_ref[jnp.maximum(b - 1, 0)]
        need_cast = (b == 0) | (bexp_ref[b] != prev)

        @pl.when(need_cast)
        def _cast():
            w1c_ref[...] = w1_ref[0].astype(jnp.bfloat16)
            w3c_ref[...] = w3_ref[0].astype(jnp.bfloat16)
            w2c_ref[...] = w2_ref[0].astype(jnp.bfloat16)

        base = b * BLK_M
        xs = xs_ref[pl.ds(base, BLK_M), :].astype(jnp.bfloat16)
        h1 = jax.lax.dot_general(
            xs, w1c_ref[...], (((1,), (0,)), ((), ())),
            preferred_element_type=jnp.float32)
        h3 = jax.lax.dot_general(
            xs, w3c_ref[...], (((1,), (0,)), ((), ())),
            preferred_element_type=jnp.float32)
        g = (h1 * jax.lax.logistic(h1)) * h3
        g = g * roww_ref[...]
        contrib = jax.lax.dot_general(
            g.astype(jnp.bfloat16), w2c_ref[...], (((1,), (0,)), ((), ())),
            preferred_element_type=jnp.float32)

        @pl.when(f == 0)
        def _set():
            acc_ref[pl.ds(base, BLK_M), :] = contrib

        @pl.when((f > 0) & (f < NF - 1))
        def _add():
            acc_ref[pl.ds(base, BLK_M), :] += contrib

        @pl.when(f == NF - 1)
        def _out():
            o_ref[...] = acc_ref[pl.ds(base, BLK_M), :] + contrib


def _ffn(xs_sorted, w1, w3, w2, row_w, bexp, bact):
    grid_spec = pltpu.PrefetchScalarGridSpec(
        num_scalar_prefetch=2,
        grid=(NF, NB_MAX),
        in_specs=[
            pl.BlockSpec((R_PAD, D), lambda f, b, be, ba: (0, 0)),
            pl.BlockSpec((1, D, BLK_F), lambda f, b, be, ba: (be[b], 0, f)),
            pl.BlockSpec((1, D, BLK_F), lambda f, b, be, ba: (be[b], 0, f)),
            pl.BlockSpec((1, BLK_F, D), lambda f, b, be, ba: (be[b], f, 0)),
            pl.BlockSpec((BLK_M, 1), lambda f, b, be, ba: (b, 0)),
        ],
        out_specs=pl.BlockSpec(
            (BLK_M, D),
            lambda f, b, be, ba: (jnp.where(f == NF - 1, b, NB_MAX), 0)),
        scratch_shapes=[
            pltpu.VMEM((R_PAD, D), jnp.float32),
            pltpu.VMEM((D, BLK_F), jnp.bfloat16),
            pltpu.VMEM((D, BLK_F), jnp.bfloat16),
            pltpu.VMEM((BLK_F, D), jnp.bfloat16),
        ],
    )
    return pl.pallas_call(
        _ffn_body,
        grid_spec=grid_spec,
        out_shape=jax.ShapeDtypeStruct(((NB_MAX + 1) * BLK_M, D), jnp.float32),
        compiler_params=pltpu.CompilerParams(
            dimension_semantics=("arbitrary", "arbitrary")),
    )(bexp, bact, xs_sorted, w1, w3, w2, row_w)


_vector_mesh = None


def _get_vector_mesh():
    global _vector_mesh
    if _vector_mesh is None:
        _vector_mesh = plsc.VectorSubcoreMesh(
            core_axis_name="core", subcore_axis_name="subcore")
    return _vector_mesh


def _sc_gather(x, row_src):
    @pl.kernel(out_type=jax.ShapeDtypeStruct((R_PAD, D), x.dtype),
               mesh=_get_vector_mesh(),
               scratch_types=[pltpu.VMEM((128,), jnp.int32),
                              pltpu.VMEM((16, D), x.dtype)])
    def k(x_hbm, i_hbm, o_hbm, idx_v, buf):
        wid = (jax.lax.axis_index("subcore") * 2
               + jax.lax.axis_index("core"))

        @pl.when(wid < 16)
        def _():
            pltpu.sync_copy(i_hbm.at[pl.ds(wid * 128, 128)], idx_v)

            @pl.loop(0, 8)
            def _(k2):
                pltpu.sync_copy(x_hbm.at[idx_v.at[pl.ds(k2 * 16, 16)]], buf)
                pltpu.sync_copy(
                    buf, o_hbm.at[pl.ds(wid * 128 + k2 * 16, 16), :])

    return k(x, row_src)


def _sc_combine(rows, pos0, pos1):
    @pl.kernel(out_type=jax.ShapeDtypeStruct((T, D), jnp.float32),
               mesh=_get_vector_mesh(),
               scratch_types=[pltpu.VMEM((T,), jnp.int32),
                              pltpu.VMEM((T,), jnp.int32),
                              pltpu.VMEM((16, D), jnp.float32),
                              pltpu.VMEM((16, D), jnp.float32)])
    def k(r_hbm, i0_hbm, i1_hbm, o_hbm, i0_v, i1_v, buf_a, buf_b):
        wid = (jax.lax.axis_index("subcore") * 2
               + jax.lax.axis_index("core"))
        pltpu.sync_copy(i0_hbm, i0_v)
        pltpu.sync_copy(i1_hbm, i1_v)
        pltpu.sync_copy(r_hbm.at[i0_v.at[pl.ds(wid * 16, 16)]], buf_a)
        pltpu.sync_copy(r_hbm.at[i1_v.at[pl.ds(wid * 16, 16)]], buf_b)

        @pl.loop(0, 16)
        def _(r):
            @pl.loop(0, D, step=16)
            def _(c):
                buf_a.at[r, pl.ds(c, 16)][...] = (
                    buf_a.at[r, pl.ds(c, 16)][...]
                    + buf_b.at[r, pl.ds(c, 16)][...])

        pltpu.sync_copy(buf_a, o_hbm.at[pl.ds(wid * 16, 16), :])

    return k(rows, pos0, pos1)


@jax.jit
def kernel(hidden_states, Wg, W1, W2, W3):
    router_logits = hidden_states @ Wg
    logits_pad = jnp.pad(router_logits, ((0, 0), (0, EPAD - E)),
                         constant_values=-jnp.inf)
    ew_t = _gate(logits_pad)
    row_src, row_w, pos0, pos1, bexp, bact = _routing_metadata(ew_t)
    xs_sorted = _sc_gather(hidden_states, row_src)

    out_rows = _ffn(xs_sorted, W1, W3, W2, row_w, bexp, bact)
    return _sc_combine(out_rows[:R_PAD], pos0, pos1)


# speedup vs baseline: 1.3402x; 1.3402x over previous
import jax
import jax.numpy as jnp
from jax.experimental import pallas as pl
from jax.experimental.pallas import tpu as pltpu
from jax.experimental.pallas import tpu_sc as plsc

T = 512
D = 2048
F = 7168
E = 8
EPAD = 128
BLK_F = 256
NF = F // BLK_F
BLK_M = 128
NB_MAX = 16
R_PAD = NB_MAX * BLK_M


def _gate_body(logits_ref, ew_ref):
    lane = jax.lax.broadcasted_iota(jnp.int32, (T, EPAD), 1)
    valid = lane < E
    neg_inf = jnp.float32(-jnp.inf)
    logits = jnp.where(valid, logits_ref[...], neg_inf)
    lmax = jnp.max(logits, axis=1, keepdims=True)
    unnorm = jnp.exp(logits - lmax)
    p = unnorm / jnp.sum(unnorm, axis=1, keepdims=True)
    p = jnp.where(valid, p, neg_inf)
    m1 = jnp.max(p, axis=1, keepdims=True)
    i1 = jnp.min(jnp.where(p == m1, lane, EPAD), axis=1, keepdims=True)
    oh1 = lane == i1
    p2 = jnp.where(oh1, neg_inf, p)
    m2 = jnp.max(p2, axis=1, keepdims=True)
    i2 = jnp.min(jnp.where(p2 == m2, lane, EPAD), axis=1, keepdims=True)
    oh2 = lane == i2
    denom = m1 + m2
    ew_ref[...] = (jnp.where(oh1, m1, 0.0) + jnp.where(oh2, m2, 0.0)) / denom


def _gate(logits_pad):
    return pl.pallas_call(
        _gate_body,
        out_shape=jax.ShapeDtypeStruct((T, EPAD), jnp.float32),
    )(logits_pad)


def _routing_metadata(ew_t):
    i32 = jnp.int32
    rw, sel = jax.lax.top_k(ew_t[:, :E], 2)
    flat_sel = sel.reshape(-1)
    order = jnp.argsort(flat_sel, stable=True)
    sorted_e = flat_sel[order]
    counts = jnp.sum(flat_sel[None, :] == jnp.arange(E)[:, None], axis=1)
    nblk = ((counts + BLK_M - 1) // BLK_M).astype(i32)
    bstart = jnp.concatenate([jnp.zeros(1, i32),
                              jnp.cumsum(nblk)[:-1].astype(i32)])
    off = jnp.concatenate([jnp.zeros(1, i32),
                           jnp.cumsum(counts)[:-1].astype(i32)])
    j = jnp.arange(2 * T, dtype=i32)
    rank = j - off[sorted_e]
    padded_pos = bstart[sorted_e] * BLK_M + rank
    row_src = jnp.zeros(R_PAD, i32).at[padded_pos].set((order // 2).astype(i32))
    row_w = jnp.zeros(R_PAD, jnp.float32).at[padded_pos].set(rw.reshape(-1)[order])
    pos_flat = jnp.zeros(2 * T, i32).at[order].set(padded_pos.astype(i32))
    pos0, pos1 = pos_flat[0::2], pos_flat[1::2]
    return row_src[:, None], row_w[:, None], pos0, pos1, nblk, bstart


def _ffn_body(nblk_ref, bstart_ref, x_ref, rsrc_ref, roww_ref,
              w1_ref, w3_ref, w2_ref, o_ref):
    e = pl.program_id(0)
    f = pl.program_id(1)
    w1 = w1_ref[0].astype(jnp.bfloat16)
    w3 = w3_ref[0].astype(jnp.bfloat16)
    w2 = w2_ref[0].astype(jnp.bfloat16)
    x = x_ref[...]

    def blk(jdx, carry):
        base = (bstart_ref[e] + jdx) * BLK_M
        idx = rsrc_ref[pl.ds(base, BLK_M), :]
        tok = jax.lax.broadcasted_iota(jnp.int32, (BLK_M, T), 1)
        oh = (tok == idx).astype(jnp.bfloat16)
        xs = jax.lax.dot_general(
            oh, x, (((1,), (0,)), ((), ())),
            preferred_element_type=jnp.float32).astype(jnp.bfloat16)
        h1 = jax.lax.dot_general(
            xs, w1, (((1,), (0,)), ((), ())),
            preferred_element_type=jnp.float32)
        h3 = jax.lax.dot_general(
            xs, w3, (((1,), (0,)), ((), ())),
            preferred_element_type=jnp.float32)
        g = (h1 * jax.lax.logistic(h1)) * h3
        g = g * roww_ref[pl.ds(base, BLK_M), :]
        contrib = jax.lax.dot_general(
            g.astype(jnp.bfloat16), w2, (((1,), (0,)), ((), ())),
            preferred_element_type=jnp.float32)

        @pl.when(f == 0)
        def _set():
            o_ref[pl.ds(base, BLK_M), :] = contrib

        @pl.when(f != 0)
        def _add():
            o_ref[pl.ds(base, BLK_M), :] += contrib

        return carry

    jax.lax.fori_loop(0, nblk_ref[e], blk, 0)


def _ffn(x_bf16, row_src, row_w, w1, w3, w2, nblk, bstart):
    grid_spec = pltpu.PrefetchScalarGridSpec(
        num_scalar_prefetch=2,
        grid=(E, NF),
        in_specs=[
            pl.BlockSpec((T, D), lambda e, f, nb, bs: (0, 0)),
            pl.BlockSpec((R_PAD, 1), lambda e, f, nb, bs: (0, 0)),
            pl.BlockSpec((R_PAD, 1), lambda e, f, nb, bs: (0, 0)),
            pl.BlockSpec((1, D, BLK_F), lambda e, f, nb, bs: (e, 0, f)),
            pl.BlockSpec((1, D, BLK_F), lambda e, f, nb, bs: (e, 0, f)),
            pl.BlockSpec((1, BLK_F, D), lambda e, f, nb, bs: (e, f, 0)),
        ],
        out_specs=pl.BlockSpec((R_PAD, D), lambda e, f, nb, bs: (0, 0)),
    )
    return pl.pallas_call(
        _ffn_body,
        grid_spec=grid_spec,
        out_shape=jax.ShapeDtypeStruct((R_PAD, D), jnp.float32),
        compiler_params=pltpu.CompilerParams(
            dimension_semantics=("arbitrary", "arbitrary")),
    )(nblk, bstart, x_bf16, row_src, row_w, w1, w3, w2)


_vector_mesh = None


def _get_vector_mesh():
    global _vector_mesh
    if _vector_mesh is None:
        _vector_mesh = plsc.VectorSubcoreMesh(
            core_axis_name="core", subcore_axis_name="subcore")
    return _vector_mesh


def _sc_combine(rows, pos0, pos1):
    @pl.kernel(out_type=jax.ShapeDtypeStruct((T, D), jnp.float32),
               mesh=_get_vector_mesh(),
               scratch_types=[pltpu.VMEM((T,), jnp.int32),
                              pltpu.VMEM((T,), jnp.int32),
                              pltpu.VMEM((16, D), jnp.float32),
                              pltpu.VMEM((16, D), jnp.float32)])
    def k(r_hbm, i0_hbm, i1_hbm, o_hbm, i0_v, i1_v, buf_a, buf_b):
        wid = (jax.lax.axis_index("subcore") * 2
               + jax.lax.axis_index("core"))
        pltpu.sync_copy(i0_hbm, i0_v)
        pltpu.sync_copy(i1_hbm, i1_v)
        pltpu.sync_copy(r_hbm.at[i0_v.at[pl.ds(wid * 16, 16)]], buf_a)
        pltpu.sync_copy(r_hbm.at[i1_v.at[pl.ds(wid * 16, 16)]], buf_b)

        @pl.loop(0, 16)
        def _(r):
            @pl.loop(0, D, step=16)
            def _(c):
                buf_a.at[r, pl.ds(c, 16)][...] = (
                    buf_a.at[r, pl.ds(c, 16)][...]
                    + buf_b.at[r, pl.ds(c, 16)][...])

        pltpu.sync_copy(buf_a, o_hbm.at[pl.ds(wid * 16, 16), :])

    return k(rows, pos0, pos1)


@jax.jit
def kernel(hidden_states, Wg, W1, W2, W3):
    router_logits = hidden_states @ Wg
    logits_pad = jnp.pad(router_logits, ((0, 0), (0, EPAD - E)),
                         constant_values=-jnp.inf)
    ew_t = _gate(logits_pad)
    row_src, row_w, pos0, pos1, nblk, bstart = _routing_metadata(ew_t)
    out_rows = _ffn(hidden_states.astype(jnp.bfloat16), row_src, row_w,
                    W1, W3, W2, nblk, bstart)
    return _sc_combine(out_rows, pos0, pos1)
